# Initial kernel scaffold; baseline (speedup 1.0000x reference)
import jax
import jax.numpy as jnp
from jax.experimental import pallas as pl


def kernel(user_table, item_table, all_top_ids, all_bottom_ids, all_users_ids, uj_rows, uj_cols, uj_vals, ij_rows, ij_cols, ij_vals):
    raise NotImplementedError("write your pallas kernel here")


# speedup vs baseline: 3.3060x; 3.3060x over previous
import functools

import jax
import jax.numpy as jnp
from jax import lax
from jax.experimental import pallas as pl
from jax.experimental.pallas import tpu as pltpu
from jax.experimental.pallas import tpu_sc as plsc

NU = 10000
NT = 10000
NB = 10000
IV = 20000
EMB = 128
NNZ = 320000

NTILES = 16
CHUNK = 128
CPT = 157
PE = CPT * CHUNK * NTILES
EPT = CPT * CHUNK

SEG = 10240
NCAT = 3 * SEG + 2048
NOUT = 10240
ROWS_PER_TILE = NOUT // NTILES
FL = 128


def _body(big_hbm, idcat_hbm, bids_hbm, tids_hbm, uids_hbm,
          ujr_hbm, ujc_hbm, ujv_hbm, ijr_hbm, ijc_hbm, ijv_hbm,
          embcat_hbm, u_hbm, t_hbm, b_hbm,
          ids_v, cols_v, ridx_v, cidx_v, vals_v, rows_v, zbuf, acc, sem):
    cid = lax.axis_index("c")
    sid = lax.axis_index("s")
    wid = sid * 2 + cid

    def emb_chunk(j, carry):
        off = (wid * 8 + j) * CHUNK
        pltpu.sync_copy(idcat_hbm.at[pl.ds(off, CHUNK)], cols_v)
        pltpu.async_copy(big_hbm.at[cols_v], rows_v, sem).wait()
        pltpu.sync_copy(rows_v, embcat_hbm.at[pl.ds(off, CHUNK)])
        return carry
    lax.fori_loop(0, 8, emb_chunk, 0)

    def zb(i, carry):
        for g in range(8):
            zbuf[i, pl.ds(g * 16, 16)] = jnp.zeros((16,), jnp.float32)
        return carry
    lax.fori_loop(0, FL, zb, 0)

    base = sid * ROWS_PER_TILE

    def zero_acc():
        for j in range(5):
            pltpu.sync_copy(zbuf, acc.at[pl.ds(base + j * FL, FL)])

    def spmm(rows_hbm, cols_hbm, vals_hbm, idtab_hbm):
        pltpu.sync_copy(idtab_hbm, ids_v)
        tile_base = sid * EPT

        def chunk(c, carry):
            off = tile_base + c * CHUNK
            pltpu.sync_copy(cols_hbm.at[pl.ds(off, CHUNK)], cols_v)
            pltpu.sync_copy(rows_hbm.at[pl.ds(off, CHUNK)], ridx_v)
            pltpu.sync_copy(vals_hbm.at[pl.ds(off, CHUNK)], vals_v)
            for j in range(8):
                c16 = cols_v[pl.ds(j * 16, 16)]
                cidx_v[pl.ds(j * 16, 16)] = plsc.load_gather(ids_v, [c16])
            pltpu.async_copy(big_hbm.at[cidx_v], rows_v, sem).wait()

            def scale(e, carry2):
                vbc = plsc.load_gather(vals_v, [jnp.full((16,), e, jnp.int32)])
                for g in range(8):
                    sl = pl.ds(g * 16, 16)
                    rows_v[e, sl] = rows_v[e, sl] * vbc
                return carry2
            lax.fori_loop(0, CHUNK, scale, 0)
            pltpu.sync_copy(rows_v, acc.at[ridx_v], add=True)
            return carry
        lax.fori_loop(0, CPT, chunk, 0)

    def flush(out_hbm):
        for j in range(5):
            pltpu.sync_copy(acc.at[pl.ds(base + j * FL, FL)],
                            rows_v.at[pl.ds(0, FL)])
            pltpu.sync_copy(rows_v.at[pl.ds(0, FL)],
                            out_hbm.at[pl.ds(base + j * FL, FL)])

    zero_acc()
    plsc.subcore_barrier()

    @pl.when(cid == 0)
    def _():
        spmm(ujr_hbm, ujc_hbm, ujv_hbm, bids_hbm)

    @pl.when(cid == 1)
    def _():
        spmm(ujc_hbm, ujr_hbm, ujv_hbm, uids_hbm)

    plsc.subcore_barrier()

    @pl.when(cid == 0)
    def _():
        flush(u_hbm)

    plsc.subcore_barrier()

    @pl.when(cid == 0)
    def _():
        zero_acc()

    plsc.subcore_barrier()

    @pl.when(cid == 0)
    def _():
        spmm(ijr_hbm, ijc_hbm, ijv_hbm, bids_hbm)

    @pl.when(cid == 1)
    def _():
        spmm(ijc_hbm, ijr_hbm, ijv_hbm, tids_hbm)

    plsc.subcore_barrier()

    @pl.when(cid == 0)
    def _():
        flush(t_hbm)

    @pl.when(cid == 1)
    def _():
        flush(b_hbm)


@jax.jit
def _run(big, idcat, bids, tids, uids, ujr, ujc, ujv, ijr, ijc, ijv):
    f32 = jnp.float32
    mesh = plsc.VectorSubcoreMesh(core_axis_name="c", subcore_axis_name="s")
    kfn = pl.kernel(
        _body,
        out_type=[
            jax.ShapeDtypeStruct((NCAT, EMB), f32),
            jax.ShapeDtypeStruct((NOUT, EMB), f32),
            jax.ShapeDtypeStruct((NOUT, EMB), f32),
            jax.ShapeDtypeStruct((NOUT, EMB), f32),
        ],
        mesh=mesh,
        scratch_types=[
            pltpu.VMEM((NU,), jnp.int32),
            pltpu.VMEM((CHUNK,), jnp.int32),
            pltpu.VMEM((CHUNK,), jnp.int32),
            pltpu.VMEM((CHUNK,), jnp.int32),
            pltpu.VMEM((CHUNK,), f32),
            pltpu.VMEM((CHUNK, EMB), f32),
            pltpu.VMEM((FL, EMB), f32),
            pltpu.VMEM_SHARED((NOUT, EMB), f32),
            pltpu.SemaphoreType.DMA,
        ],
        compiler_params=pltpu.CompilerParams(needs_layout_passes=False),
    )
    return kfn(big, idcat, bids, tids, uids, ujr, ujc, ujv, ijr, ijc, ijv)


def kernel(user_table, item_table, all_top_ids, all_bottom_ids, all_users_ids,
           uj_rows, uj_cols, uj_vals, ij_rows, ij_cols, ij_vals):
    i32 = jnp.int32
    big = jnp.concatenate([item_table, user_table], axis=0)
    tids = all_top_ids.astype(i32)
    bids = all_bottom_ids.astype(i32)
    uids = all_users_ids.astype(i32) + IV

    idcat = jnp.zeros((NCAT,), i32)
    idcat = lax.dynamic_update_slice(idcat, tids, (0,))
    idcat = lax.dynamic_update_slice(idcat, bids, (SEG,))
    idcat = lax.dynamic_update_slice(idcat, uids, (2 * SEG,))

    zi = jnp.zeros((PE,), i32)
    zf = jnp.zeros((PE,), jnp.float32)
    ujr = lax.dynamic_update_slice(zi, uj_rows.astype(i32), (0,))
    ujc = lax.dynamic_update_slice(zi, uj_cols.astype(i32), (0,))
    ujv = lax.dynamic_update_slice(zf, uj_vals, (0,))
    ijr = lax.dynamic_update_slice(zi, ij_rows.astype(i32), (0,))
    ijc = lax.dynamic_update_slice(zi, ij_cols.astype(i32), (0,))
    ijv = lax.dynamic_update_slice(zf, ij_vals, (0,))

    embcat, u_out, t_out, b_out = _run(big, idcat, bids, tids, uids,
                                       ujr, ujc, ujv, ijr, ijc, ijv)
    top_embs = embcat[0:NT]
    pos_bottoms_embs = embcat[SEG:SEG + NB]
    all_users_embs = embcat[2 * SEG:2 * SEG + NU]
    return (u_out[:NU], t_out[:NT], b_out[:NB],
            top_embs, pos_bottoms_embs, all_users_embs)


# speedup vs baseline: 4.4912x; 1.3585x over previous
import jax
import jax.numpy as jnp
from jax import lax
from jax.experimental import pallas as pl
from jax.experimental.pallas import tpu as pltpu
from jax.experimental.pallas import tpu_sc as plsc

NU = 10000
NT = 10000
NB = 10000
IV = 20000
EMB = 128
NNZ = 320000

NTILES = 16
CHUNK = 128
CPT = 158
PE = CPT * CHUNK * NTILES
PKW = 3 * CHUNK

SEG = 10240
NCAT = 3 * SEG + 2048
NOUT = 10240
ROWS_PER_TILE = NOUT // NTILES
FL = 128


def _body(big_hbm, idcat_hbm, bids_hbm, tids_hbm, uids_hbm,
          pka_hbm, pkb_hbm, pkc_hbm, pkd_hbm,
          embcat_hbm, u_hbm, t_hbm, b_hbm,
          ids_v, idx0, idx1, cidx0, cidx1, ridx0, ridx1,
          rows0, rows1, acc, sem0, sem1):
    cid = lax.axis_index("c")
    sid = lax.axis_index("s")
    wid = sid * 2 + cid
    idx_b = (idx0, idx1)
    cidx_b = (cidx0, cidx1)
    ridx_b = (ridx0, ridx1)
    rows_b = (rows0, rows1)
    sem_b = (sem0, sem1)

    def emb_chunk(j, carry):
        off = (wid * 8 + j) * CHUNK
        pltpu.sync_copy(idcat_hbm.at[pl.ds(off, CHUNK)], cidx0)
        pltpu.async_copy(big_hbm.at[cidx0], rows0, sem0).wait()
        pltpu.sync_copy(rows0, embcat_hbm.at[pl.ds(off, CHUNK)])
        return carry
    lax.fori_loop(0, 8, emb_chunk, 0)

    base = sid * ROWS_PER_TILE

    def zero_acc():
        def zb(i, carry):
            for g in range(8):
                rows0[i, pl.ds(g * 16, 16)] = jnp.zeros((16,), jnp.float32)
            return carry
        lax.fori_loop(0, FL, zb, 0)
        for j in range(5):
            pltpu.sync_copy(rows0, acc.at[pl.ds(base + j * FL, FL)])

    def spmm(pk_hbm):
        tile_chunk0 = sid * CPT

        def prep(c, b):
            off = (tile_chunk0 + c) * PKW
            pltpu.sync_copy(pk_hbm.at[pl.ds(off, PKW)], idx_b[b])
            for j in range(8):
                sl = pl.ds(j * 16, 16)
                c16 = idx_b[b][sl]
                cidx_b[b][sl] = plsc.load_gather(ids_v, [c16])
                ridx_b[b][sl] = idx_b[b][pl.ds(CHUNK + j * 16, 16)]
            return pltpu.async_copy(big_hbm.at[cidx_b[b]], rows_b[b], sem_b[b])

        prep(0, 0)

        def outer(i, carry):
            for b in range(2):
                c = 2 * i + b
                cn = c + 1
                if b == 0:
                    prep(cn, 1)
                else:
                    @pl.when(cn < CPT)
                    def _():
                        prep(cn, 0)
                pltpu.make_async_copy(big_hbm.at[cidx_b[b]],
                                      rows_b[b], sem_b[b]).wait()

                def scale(e, c2):
                    vb = plsc.bitcast(
                        plsc.load_gather(
                            idx_b[b], [jnp.full((16,), 2 * CHUNK + e,
                                                jnp.int32)]),
                        jnp.float32)
                    for g in range(8):
                        sl = pl.ds(g * 16, 16)
                        rows_b[b][e, sl] = rows_b[b][e, sl] * vb
                    return c2
                lax.fori_loop(0, CHUNK, scale, 0)

                pltpu.sync_copy(rows_b[b], acc.at[ridx_b[b]], add=True)
            return carry
        lax.fori_loop(0, CPT // 2, outer, 0)

    def flush(out_hbm):
        for j in range(5):
            pltpu.sync_copy(acc.at[pl.ds(base + j * FL, FL)], rows0)
            pltpu.sync_copy(rows0, out_hbm.at[pl.ds(base + j * FL, FL)])

    zero_acc()
    plsc.subcore_barrier()

    @pl.when(cid == 0)
    def _():
        pltpu.sync_copy(bids_hbm, ids_v)
        spmm(pka_hbm)

    @pl.when(cid == 1)
    def _():
        pltpu.sync_copy(uids_hbm, ids_v)
        spmm(pkb_hbm)

    plsc.subcore_barrier()

    @pl.when(cid == 0)
    def _():
        flush(u_hbm)

    plsc.subcore_barrier()

    @pl.when(cid == 0)
    def _():
        zero_acc()

    plsc.subcore_barrier()

    @pl.when(cid == 0)
    def _():
        spmm(pkc_hbm)

    @pl.when(cid == 1)
    def _():
        pltpu.sync_copy(tids_hbm, ids_v)
        spmm(pkd_hbm)

    plsc.subcore_barrier()

    @pl.when(cid == 0)
    def _():
        flush(t_hbm)

    @pl.when(cid == 1)
    def _():
        flush(b_hbm)


@jax.jit
def _run(big, idcat, bids, tids, uids, pka, pkb, pkc, pkd):
    f32 = jnp.float32
    i32 = jnp.int32
    mesh = plsc.VectorSubcoreMesh(core_axis_name="c", subcore_axis_name="s")
    kfn = pl.kernel(
        _body,
        out_type=[
            jax.ShapeDtypeStruct((NCAT, EMB), f32),
            jax.ShapeDtypeStruct((NOUT, EMB), f32),
            jax.ShapeDtypeStruct((NOUT, EMB), f32),
            jax.ShapeDtypeStruct((NOUT, EMB), f32),
        ],
        mesh=mesh,
        scratch_types=[
            pltpu.VMEM((NU,), i32),
            pltpu.VMEM((PKW,), i32),
            pltpu.VMEM((PKW,), i32),
            pltpu.VMEM((CHUNK,), i32),
            pltpu.VMEM((CHUNK,), i32),
            pltpu.VMEM((CHUNK,), i32),
            pltpu.VMEM((CHUNK,), i32),
            pltpu.VMEM((CHUNK, EMB), f32),
            pltpu.VMEM((CHUNK, EMB), f32),
            pltpu.VMEM_SHARED((NOUT, EMB), f32),
            pltpu.SemaphoreType.DMA,
            pltpu.SemaphoreType.DMA,
        ],
        compiler_params=pltpu.CompilerParams(needs_layout_passes=False),
    )
    return kfn(big, idcat, bids, tids, uids, pka, pkb, pkc, pkd)


def _pack(cols, rows, vals):
    i32 = jnp.int32
    nch = PE // CHUNK
    zi = jnp.zeros((PE,), i32)
    c = lax.dynamic_update_slice(zi, cols.astype(i32), (0,)).reshape(nch, CHUNK)
    r = lax.dynamic_update_slice(zi, rows.astype(i32), (0,)).reshape(nch, CHUNK)
    v = lax.dynamic_update_slice(
        zi, lax.bitcast_convert_type(vals, i32), (0,)).reshape(nch, CHUNK)
    return jnp.stack([c, r, v], axis=1).reshape(-1)


def kernel(user_table, item_table, all_top_ids, all_bottom_ids, all_users_ids,
           uj_rows, uj_cols, uj_vals, ij_rows, ij_cols, ij_vals):
    i32 = jnp.int32
    big = jnp.concatenate([item_table, user_table], axis=0)
    tids = all_top_ids.astype(i32)
    bids = all_bottom_ids.astype(i32)
    uids = all_users_ids.astype(i32) + IV

    idcat = jnp.zeros((NCAT,), i32)
    idcat = lax.dynamic_update_slice(idcat, tids, (0,))
    idcat = lax.dynamic_update_slice(idcat, bids, (SEG,))
    idcat = lax.dynamic_update_slice(idcat, uids, (2 * SEG,))

    pka = _pack(uj_cols, uj_rows, uj_vals)
    pkb = _pack(uj_rows, uj_cols, uj_vals)
    pkc = _pack(ij_cols, ij_rows, ij_vals)
    pkd = _pack(ij_rows, ij_cols, ij_vals)

    embcat, u_out, t_out, b_out = _run(big, idcat, bids, tids, uids,
                                       pka, pkb, pkc, pkd)
    top_embs = embcat[0:NT]
    pos_bottoms_embs = embcat[SEG:SEG + NB]
    all_users_embs = embcat[2 * SEG:2 * SEG + NU]
    return (u_out[:NU], t_out[:NT], b_out[:NB],
            top_embs, pos_bottoms_embs, all_users_embs)


# speedup vs baseline: 5.1207x; 1.1402x over previous
import jax
import jax.numpy as jnp
from jax import lax
from jax.experimental import pallas as pl
from jax.experimental.pallas import tpu as pltpu
from jax.experimental.pallas import tpu_sc as plsc

NU = 10000
NT = 10000
NB = 10000
IV = 20000
EMB = 128
NNZ = 320000

NTILES = 16
CHUNK = 128
CPT = 158
PE = CPT * CHUNK * NTILES
PKW = 3 * CHUNK

SEG = 10240
NCAT = 3 * SEG + 2048
NOUT = 10240
ROWS_PER_TILE = NOUT // NTILES
FL = 128


def _body(big_hbm, idcat_hbm, bids_hbm, tids_hbm, uids_hbm,
          pka_hbm, pkb_hbm, pkc_hbm, pkd_hbm,
          embcat_hbm, u_hbm, t_hbm, b_hbm,
          ids_v, idx0, idx1, cidx0, cidx1, ridx0, ridx1,
          rows0, rows1, acc, sem0, sem1, ssem0, ssem1):
    cid = lax.axis_index("c")
    sid = lax.axis_index("s")
    wid = sid * 2 + cid
    idx_b = (idx0, idx1)
    cidx_b = (cidx0, cidx1)
    ridx_b = (ridx0, ridx1)
    rows_b = (rows0, rows1)
    sem_b = (sem0, sem1)
    ssem_b = (ssem0, ssem1)

    def emb_chunk(j, carry):
        off = (wid * 8 + j) * CHUNK
        pltpu.sync_copy(idcat_hbm.at[pl.ds(off, CHUNK)], cidx0)
        pltpu.async_copy(big_hbm.at[cidx0], rows0, sem0).wait()
        pltpu.sync_copy(rows0, embcat_hbm.at[pl.ds(off, CHUNK)])
        return carry
    lax.fori_loop(0, 8, emb_chunk, 0)

    base = sid * ROWS_PER_TILE

    def zero_acc():
        def zb(i, carry):
            for g in range(8):
                rows0[i, pl.ds(g * 16, 16)] = jnp.zeros((16,), jnp.float32)
            return carry
        lax.fori_loop(0, FL, zb, 0)
        for j in range(5):
            pltpu.sync_copy(rows0, acc.at[pl.ds(base + j * FL, FL)])

    def spmm(pk_hbm):
        tile_chunk0 = sid * CPT

        def prep(c, b):
            @pl.when(c >= 2)
            def _():
                pltpu.make_async_copy(rows_b[b], acc.at[ridx_b[b]],
                                      ssem_b[b]).wait()
            off = (tile_chunk0 + c) * PKW
            pltpu.sync_copy(pk_hbm.at[pl.ds(off, PKW)], idx_b[b])
            for j in range(8):
                sl = pl.ds(j * 16, 16)
                c16 = idx_b[b][sl]
                cidx_b[b][sl] = plsc.load_gather(ids_v, [c16])
                ridx_b[b][sl] = idx_b[b][pl.ds(CHUNK + j * 16, 16)]
            return pltpu.async_copy(big_hbm.at[cidx_b[b]], rows_b[b], sem_b[b])

        prep(0, 0)

        def outer(i, carry):
            for b in range(2):
                c = 2 * i + b
                cn = c + 1
                if b == 0:
                    prep(cn, 1)
                else:
                    @pl.when(cn < CPT)
                    def _():
                        prep(cn, 0)
                pltpu.make_async_copy(big_hbm.at[cidx_b[b]],
                                      rows_b[b], sem_b[b]).wait()

                @plsc.parallel_loop(0, CHUNK, unroll=2)
                def _(e):
                    vb = plsc.bitcast(
                        plsc.load_gather(
                            idx_b[b], [jnp.full((16,), 2 * CHUNK + e,
                                                jnp.int32)]),
                        jnp.float32)
                    for g in range(8):
                        sl = pl.ds(g * 16, 16)
                        rows_b[b][e, sl] = rows_b[b][e, sl] * vb

                pltpu.async_copy(rows_b[b], acc.at[ridx_b[b]], ssem_b[b],
                                 add=True)
            return carry
        lax.fori_loop(0, CPT // 2, outer, 0)
        for b in range(2):
            pltpu.make_async_copy(rows_b[b], acc.at[ridx_b[b]],
                                  ssem_b[b]).wait()

    def flush(out_hbm):
        for j in range(5):
            pltpu.sync_copy(acc.at[pl.ds(base + j * FL, FL)], rows0)
            pltpu.sync_copy(rows0, out_hbm.at[pl.ds(base + j * FL, FL)])

    zero_acc()
    plsc.subcore_barrier()

    @pl.when(cid == 0)
    def _():
        pltpu.sync_copy(bids_hbm, ids_v)
        spmm(pka_hbm)

    @pl.when(cid == 1)
    def _():
        pltpu.sync_copy(uids_hbm, ids_v)
        spmm(pkb_hbm)

    plsc.subcore_barrier()

    @pl.when(cid == 0)
    def _():
        flush(u_hbm)

    plsc.subcore_barrier()

    @pl.when(cid == 0)
    def _():
        zero_acc()

    plsc.subcore_barrier()

    @pl.when(cid == 0)
    def _():
        spmm(pkc_hbm)

    @pl.when(cid == 1)
    def _():
        pltpu.sync_copy(tids_hbm, ids_v)
        spmm(pkd_hbm)

    plsc.subcore_barrier()

    @pl.when(cid == 0)
    def _():
        flush(t_hbm)

    @pl.when(cid == 1)
    def _():
        flush(b_hbm)


@jax.jit
def _run(big, idcat, bids, tids, uids, pka, pkb, pkc, pkd):
    f32 = jnp.float32
    i32 = jnp.int32
    mesh = plsc.VectorSubcoreMesh(core_axis_name="c", subcore_axis_name="s")
    kfn = pl.kernel(
        _body,
        out_type=[
            jax.ShapeDtypeStruct((NCAT, EMB), f32),
            jax.ShapeDtypeStruct((NOUT, EMB), f32),
            jax.ShapeDtypeStruct((NOUT, EMB), f32),
            jax.ShapeDtypeStruct((NOUT, EMB), f32),
        ],
        mesh=mesh,
        scratch_types=[
            pltpu.VMEM((NU,), i32),
            pltpu.VMEM((PKW,), i32),
            pltpu.VMEM((PKW,), i32),
            pltpu.VMEM((CHUNK,), i32),
            pltpu.VMEM((CHUNK,), i32),
            pltpu.VMEM((CHUNK,), i32),
            pltpu.VMEM((CHUNK,), i32),
            pltpu.VMEM((CHUNK, EMB), f32),
            pltpu.VMEM((CHUNK, EMB), f32),
            pltpu.VMEM_SHARED((NOUT, EMB), f32),
            pltpu.SemaphoreType.DMA,
            pltpu.SemaphoreType.DMA,
            pltpu.SemaphoreType.DMA,
            pltpu.SemaphoreType.DMA,
        ],
        compiler_params=pltpu.CompilerParams(needs_layout_passes=False),
    )
    return kfn(big, idcat, bids, tids, uids, pka, pkb, pkc, pkd)


def _pack(cols, rows, vals):
    i32 = jnp.int32
    nch = PE // CHUNK
    zi = jnp.zeros((PE,), i32)
    c = lax.dynamic_update_slice(zi, cols.astype(i32), (0,)).reshape(nch, CHUNK)
    r = lax.dynamic_update_slice(zi, rows.astype(i32), (0,)).reshape(nch, CHUNK)
    v = lax.dynamic_update_slice(
        zi, lax.bitcast_convert_type(vals, i32), (0,)).reshape(nch, CHUNK)
    return jnp.stack([c, r, v], axis=1).reshape(-1)


def kernel(user_table, item_table, all_top_ids, all_bottom_ids, all_users_ids,
           uj_rows, uj_cols, uj_vals, ij_rows, ij_cols, ij_vals):
    i32 = jnp.int32
    big = jnp.concatenate([item_table, user_table], axis=0)
    tids = all_top_ids.astype(i32)
    bids = all_bottom_ids.astype(i32)
    uids = all_users_ids.astype(i32) + IV

    idcat = jnp.zeros((NCAT,), i32)
    idcat = lax.dynamic_update_slice(idcat, tids, (0,))
    idcat = lax.dynamic_update_slice(idcat, bids, (SEG,))
    idcat = lax.dynamic_update_slice(idcat, uids, (2 * SEG,))

    pka = _pack(uj_cols, uj_rows, uj_vals)
    pkb = _pack(uj_rows, uj_cols, uj_vals)
    pkc = _pack(ij_cols, ij_rows, ij_vals)
    pkd = _pack(ij_rows, ij_cols, ij_vals)

    embcat, u_out, t_out, b_out = _run(big, idcat, bids, tids, uids,
                                       pka, pkb, pkc, pkd)
    top_embs = embcat[0:NT]
    pos_bottoms_embs = embcat[SEG:SEG + NB]
    all_users_embs = embcat[2 * SEG:2 * SEG + NU]
    return (u_out[:NU], t_out[:NT], b_out[:NB],
            top_embs, pos_bottoms_embs, all_users_embs)


# speedup vs baseline: 5.7596x; 1.1248x over previous
import jax
import jax.numpy as jnp
from jax import lax
from jax.experimental import pallas as pl
from jax.experimental.pallas import tpu as pltpu
from jax.experimental.pallas import tpu_sc as plsc

NU = 10000
NT = 10000
NB = 10000
IV = 20000
EMB = 128
NNZ = 320000

NTILES = 16
CHUNK = 96
CPT = 210
PE = CPT * CHUNK * NTILES
PKW = 3 * CHUNK

SEG = 10240
ECPW = 11
NCAT = 32 * ECPW * CHUNK
NOUT = 10240
ROWS_PER_TILE = NOUT // NTILES
FL = 80


def _body(big_hbm, idcat_hbm, bids_hbm, tids_hbm, uids_hbm,
          pka_hbm, pkb_hbm, pkc_hbm, pkd_hbm,
          embcat_hbm, u_hbm, t_hbm, b_hbm,
          ids_v, idx0, idx1, idx2, cidx0, cidx1, cidx2,
          ridx0, ridx1, ridx2, rows0, rows1, rows2, acc,
          gsem0, gsem1, gsem2, ssem0, ssem1, ssem2, isem0, isem1, isem2):
    cid = lax.axis_index("c")
    sid = lax.axis_index("s")
    wid = sid * 2 + cid
    idx_b = (idx0, idx1, idx2)
    cidx_b = (cidx0, cidx1, cidx2)
    ridx_b = (ridx0, ridx1, ridx2)
    rows_b = (rows0, rows1, rows2)
    gsem_b = (gsem0, gsem1, gsem2)
    ssem_b = (ssem0, ssem1, ssem2)
    isem_b = (isem0, isem1, isem2)

    def emb_chunk(j, carry):
        off = (wid * ECPW + j) * CHUNK
        pltpu.sync_copy(idcat_hbm.at[pl.ds(off, CHUNK)], cidx0)
        pltpu.async_copy(big_hbm.at[cidx0], rows0, gsem0).wait()
        pltpu.sync_copy(rows0, embcat_hbm.at[pl.ds(off, CHUNK)])
        return carry
    lax.fori_loop(0, ECPW, emb_chunk, 0)

    base = sid * ROWS_PER_TILE

    def zero_acc():
        def zb(i, carry):
            for g in range(8):
                rows0[i, pl.ds(g * 16, 16)] = jnp.zeros((16,), jnp.float32)
            return carry
        lax.fori_loop(0, CHUNK, zb, 0)
        for j in range(8):
            pltpu.sync_copy(rows0.at[pl.ds(0, FL)],
                            acc.at[pl.ds(base + j * FL, FL)])

    def spmm(pk_hbm):
        tile_chunk0 = sid * CPT

        def prep_idx(c, s):
            off = (tile_chunk0 + c) * PKW
            pltpu.async_copy(pk_hbm.at[pl.ds(off, PKW)], idx_b[s], isem_b[s])

        def compose_and_gather(c, s):
            @pl.when(c >= 3)
            def _():
                pltpu.make_async_copy(rows_b[s], acc.at[ridx_b[s]],
                                      ssem_b[s]).wait()
            pltpu.make_async_copy(pk_hbm.at[pl.ds(0, PKW)], idx_b[s],
                                  isem_b[s]).wait()
            for j in range(CHUNK // 16):
                sl = pl.ds(j * 16, 16)
                c16 = idx_b[s][sl]
                cidx_b[s][sl] = plsc.load_gather(ids_v, [c16])
                ridx_b[s][sl] = idx_b[s][pl.ds(CHUNK + j * 16, 16)]
            pltpu.async_copy(big_hbm.at[cidx_b[s]], rows_b[s], gsem_b[s])

        prep_idx(0, 0)
        prep_idx(1, 1)
        compose_and_gather(0, 0)

        def outer(i, carry):
            for k in range(3):
                c = 3 * i + k
                s2 = (k + 2) % 3
                s1 = (k + 1) % 3

                @pl.when(c + 2 < CPT)
                def _():
                    prep_idx(c + 2, s2)

                @pl.when(c + 1 < CPT)
                def _():
                    compose_and_gather(c + 1, s1)

                pltpu.make_async_copy(big_hbm.at[cidx_b[k]],
                                      rows_b[k], gsem_b[k]).wait()

                @plsc.parallel_loop(0, CHUNK, unroll=2)
                def _(e):
                    vb = plsc.bitcast(
                        plsc.load_gather(
                            idx_b[k], [jnp.full((16,), 2 * CHUNK + e,
                                                jnp.int32)]),
                        jnp.float32)
                    for g in range(8):
                        sl = pl.ds(g * 16, 16)
                        rows_b[k][e, sl] = rows_b[k][e, sl] * vb

                pltpu.async_copy(rows_b[k], acc.at[ridx_b[k]], ssem_b[k],
                                 add=True)
            return carry
        lax.fori_loop(0, CPT // 3, outer, 0)
        for s in range(3):
            pltpu.make_async_copy(rows_b[s], acc.at[ridx_b[s]],
                                  ssem_b[s]).wait()

    def flush(out_hbm):
        for j in range(8):
            pltpu.sync_copy(acc.at[pl.ds(base + j * FL, FL)],
                            rows0.at[pl.ds(0, FL)])
            pltpu.sync_copy(rows0.at[pl.ds(0, FL)],
                            out_hbm.at[pl.ds(base + j * FL, FL)])

    zero_acc()
    plsc.subcore_barrier()

    @pl.when(cid == 0)
    def _():
        pltpu.sync_copy(bids_hbm, ids_v)
        spmm(pka_hbm)

    @pl.when(cid == 1)
    def _():
        pltpu.sync_copy(uids_hbm, ids_v)
        spmm(pkb_hbm)

    plsc.subcore_barrier()

    @pl.when(cid == 0)
    def _():
        flush(u_hbm)

    plsc.subcore_barrier()

    @pl.when(cid == 0)
    def _():
        zero_acc()

    plsc.subcore_barrier()

    @pl.when(cid == 0)
    def _():
        spmm(pkc_hbm)

    @pl.when(cid == 1)
    def _():
        pltpu.sync_copy(tids_hbm, ids_v)
        spmm(pkd_hbm)

    plsc.subcore_barrier()

    @pl.when(cid == 0)
    def _():
        flush(t_hbm)

    @pl.when(cid == 1)
    def _():
        flush(b_hbm)


@jax.jit
def _run(big, idcat, bids, tids, uids, pka, pkb, pkc, pkd):
    f32 = jnp.float32
    i32 = jnp.int32
    mesh = plsc.VectorSubcoreMesh(core_axis_name="c", subcore_axis_name="s")
    kfn = pl.kernel(
        _body,
        out_type=[
            jax.ShapeDtypeStruct((NCAT, EMB), f32),
            jax.ShapeDtypeStruct((NOUT, EMB), f32),
            jax.ShapeDtypeStruct((NOUT, EMB), f32),
            jax.ShapeDtypeStruct((NOUT, EMB), f32),
        ],
        mesh=mesh,
        scratch_types=[
            pltpu.VMEM((NU,), i32),
            pltpu.VMEM((PKW,), i32),
            pltpu.VMEM((PKW,), i32),
            pltpu.VMEM((PKW,), i32),
            pltpu.VMEM((CHUNK,), i32),
            pltpu.VMEM((CHUNK,), i32),
            pltpu.VMEM((CHUNK,), i32),
            pltpu.VMEM((CHUNK,), i32),
            pltpu.VMEM((CHUNK,), i32),
            pltpu.VMEM((CHUNK,), i32),
            pltpu.VMEM((CHUNK, EMB), f32),
            pltpu.VMEM((CHUNK, EMB), f32),
            pltpu.VMEM((CHUNK, EMB), f32),
            pltpu.VMEM_SHARED((NOUT, EMB), f32),
            pltpu.SemaphoreType.DMA,
            pltpu.SemaphoreType.DMA,
            pltpu.SemaphoreType.DMA,
            pltpu.SemaphoreType.DMA,
            pltpu.SemaphoreType.DMA,
            pltpu.SemaphoreType.DMA,
            pltpu.SemaphoreType.DMA,
            pltpu.SemaphoreType.DMA,
            pltpu.SemaphoreType.DMA,
        ],
        compiler_params=pltpu.CompilerParams(needs_layout_passes=False),
    )
    return kfn(big, idcat, bids, tids, uids, pka, pkb, pkc, pkd)


def _pack(cols, rows, vals):
    i32 = jnp.int32
    nch = PE // CHUNK
    zi = jnp.zeros((PE,), i32)
    c = lax.dynamic_update_slice(zi, cols.astype(i32), (0,)).reshape(nch, CHUNK)
    r = lax.dynamic_update_slice(zi, rows.astype(i32), (0,)).reshape(nch, CHUNK)
    v = lax.dynamic_update_slice(
        zi, lax.bitcast_convert_type(vals, i32), (0,)).reshape(nch, CHUNK)
    return jnp.stack([c, r, v], axis=1).reshape(-1)


def kernel(user_table, item_table, all_top_ids, all_bottom_ids, all_users_ids,
           uj_rows, uj_cols, uj_vals, ij_rows, ij_cols, ij_vals):
    i32 = jnp.int32
    big = jnp.concatenate([item_table, user_table], axis=0)
    tids = all_top_ids.astype(i32)
    bids = all_bottom_ids.astype(i32)
    uids = all_users_ids.astype(i32) + IV

    idcat = jnp.zeros((NCAT,), i32)
    idcat = lax.dynamic_update_slice(idcat, tids, (0,))
    idcat = lax.dynamic_update_slice(idcat, bids, (SEG,))
    idcat = lax.dynamic_update_slice(idcat, uids, (2 * SEG,))

    pka = _pack(uj_cols, uj_rows, uj_vals)
    pkb = _pack(uj_rows, uj_cols, uj_vals)
    pkc = _pack(ij_cols, ij_rows, ij_vals)
    pkd = _pack(ij_rows, ij_cols, ij_vals)

    embcat, u_out, t_out, b_out = _run(big, idcat, bids, tids, uids,
                                       pka, pkb, pkc, pkd)
    top_embs = embcat[0:NT]
    pos_bottoms_embs = embcat[SEG:SEG + NB]
    all_users_embs = embcat[2 * SEG:2 * SEG + NU]
    return (u_out[:NU], t_out[:NT], b_out[:NB],
            top_embs, pos_bottoms_embs, all_users_embs)


# speedup vs baseline: 5.7627x; 1.0005x over previous
import jax
import jax.numpy as jnp
from jax import lax
from jax.experimental import pallas as pl
from jax.experimental.pallas import tpu as pltpu
from jax.experimental.pallas import tpu_sc as plsc

NU = 10000
NT = 10000
NB = 10000
IV = 20000
EMB = 128
NNZ = 320000

NTILES = 16
CHUNK = 96
CPT = 210
PE = CPT * CHUNK * NTILES
PKW = 3 * CHUNK

SEG = 10240
ECPW = 11
NCAT = 32 * ECPW * CHUNK
NOUT = 10240
ROWS_PER_TILE = NOUT // NTILES
FL = 80


def _body(big_hbm, idcat_hbm, bids_hbm, tids_hbm, uids_hbm,
          pka_hbm, pkb_hbm, pkc_hbm, pkd_hbm,
          embcat_hbm, u_hbm, t_hbm, b_hbm,
          ids_v, idx0, idx1, idx2, cidx0, cidx1, cidx2,
          ridx0, ridx1, ridx2, rows0, rows1, rows2, acc,
          gsem0, gsem1, gsem2, ssem0, ssem1, ssem2, isem0, isem1, isem2):
    cid = lax.axis_index("c")
    sid = lax.axis_index("s")
    wid = sid * 2 + cid
    idx_b = (idx0, idx1, idx2)
    cidx_b = (cidx0, cidx1, cidx2)
    ridx_b = (ridx0, ridx1, ridx2)
    rows_b = (rows0, rows1, rows2)
    gsem_b = (gsem0, gsem1, gsem2)
    ssem_b = (ssem0, ssem1, ssem2)
    isem_b = (isem0, isem1, isem2)

    def emb_chunk(j, carry):
        off = (wid * ECPW + j) * CHUNK
        pltpu.sync_copy(idcat_hbm.at[pl.ds(off, CHUNK)], cidx0)
        pltpu.async_copy(big_hbm.at[cidx0], rows0, gsem0).wait()
        pltpu.sync_copy(rows0, embcat_hbm.at[pl.ds(off, CHUNK)])
        return carry
    lax.fori_loop(0, ECPW, emb_chunk, 0)

    base = sid * ROWS_PER_TILE

    def zero_acc():
        def zb(i, carry):
            for g in range(8):
                rows0[i, pl.ds(g * 16, 16)] = jnp.zeros((16,), jnp.float32)
            return carry
        lax.fori_loop(0, CHUNK, zb, 0)
        for j in range(8):
            pltpu.sync_copy(rows0.at[pl.ds(0, FL)],
                            acc.at[pl.ds(base + j * FL, FL)])

    def spmm(pk_hbm):
        tile_chunk0 = sid * CPT

        def prep_idx(c, s):
            off = (tile_chunk0 + c) * PKW
            pltpu.async_copy(pk_hbm.at[pl.ds(off, PKW)], idx_b[s], isem_b[s])

        def compose_and_gather(c, s):
            @pl.when(c >= 3)
            def _():
                pltpu.make_async_copy(rows_b[s], acc.at[ridx_b[s]],
                                      ssem_b[s]).wait()
            pltpu.make_async_copy(pk_hbm.at[pl.ds(0, PKW)], idx_b[s],
                                  isem_b[s]).wait()
            for j in range(CHUNK // 16):
                sl = pl.ds(j * 16, 16)
                c16 = idx_b[s][sl]
                cidx_b[s][sl] = plsc.load_gather(ids_v, [c16])
                ridx_b[s][sl] = idx_b[s][pl.ds(CHUNK + j * 16, 16)]
            pltpu.async_copy(big_hbm.at[cidx_b[s]], rows_b[s], gsem_b[s])

        prep_idx(0, 0)
        prep_idx(1, 1)
        compose_and_gather(0, 0)

        def outer(i, carry):
            for k in range(3):
                c = 3 * i + k
                s2 = (k + 2) % 3
                s1 = (k + 1) % 3

                @pl.when(c + 2 < CPT)
                def _():
                    prep_idx(c + 2, s2)

                @pl.when(c + 1 < CPT)
                def _():
                    compose_and_gather(c + 1, s1)

                pltpu.make_async_copy(big_hbm.at[cidx_b[k]],
                                      rows_b[k], gsem_b[k]).wait()

                @plsc.parallel_loop(0, CHUNK, unroll=4)
                def _(e):
                    vb = plsc.bitcast(
                        plsc.load_gather(
                            idx_b[k], [jnp.full((16,), 2 * CHUNK + e,
                                                jnp.int32)]),
                        jnp.float32)
                    for g in range(8):
                        sl = pl.ds(g * 16, 16)
                        rows_b[k][e, sl] = rows_b[k][e, sl] * vb

                pltpu.async_copy(rows_b[k], acc.at[ridx_b[k]], ssem_b[k],
                                 add=True)
            return carry
        lax.fori_loop(0, CPT // 3, outer, 0)
        for s in range(3):
            pltpu.make_async_copy(rows_b[s], acc.at[ridx_b[s]],
                                  ssem_b[s]).wait()

    def flush(out_hbm):
        for j in range(8):
            pltpu.sync_copy(acc.at[pl.ds(base + j * FL, FL)],
                            rows0.at[pl.ds(0, FL)])
            pltpu.sync_copy(rows0.at[pl.ds(0, FL)],
                            out_hbm.at[pl.ds(base + j * FL, FL)])

    zero_acc()
    plsc.subcore_barrier()

    @pl.when(cid == 0)
    def _():
        pltpu.sync_copy(bids_hbm, ids_v)
        spmm(pka_hbm)

    @pl.when(cid == 1)
    def _():
        pltpu.sync_copy(uids_hbm, ids_v)
        spmm(pkb_hbm)

    plsc.subcore_barrier()

    @pl.when(cid == 0)
    def _():
        flush(u_hbm)

    plsc.subcore_barrier()

    @pl.when(cid == 0)
    def _():
        zero_acc()

    plsc.subcore_barrier()

    @pl.when(cid == 0)
    def _():
        spmm(pkc_hbm)

    @pl.when(cid == 1)
    def _():
        pltpu.sync_copy(tids_hbm, ids_v)
        spmm(pkd_hbm)

    plsc.subcore_barrier()

    @pl.when(cid == 0)
    def _():
        flush(t_hbm)

    @pl.when(cid == 1)
    def _():
        flush(b_hbm)


@jax.jit
def _run(big, idcat, bids, tids, uids, pka, pkb, pkc, pkd):
    f32 = jnp.float32
    i32 = jnp.int32
    mesh = plsc.VectorSubcoreMesh(core_axis_name="c", subcore_axis_name="s")
    kfn = pl.kernel(
        _body,
        out_type=[
            jax.ShapeDtypeStruct((NCAT, EMB), f32),
            jax.ShapeDtypeStruct((NOUT, EMB), f32),
            jax.ShapeDtypeStruct((NOUT, EMB), f32),
            jax.ShapeDtypeStruct((NOUT, EMB), f32),
        ],
        mesh=mesh,
        scratch_types=[
            pltpu.VMEM((NU,), i32),
            pltpu.VMEM((PKW,), i32),
            pltpu.VMEM((PKW,), i32),
            pltpu.VMEM((PKW,), i32),
            pltpu.VMEM((CHUNK,), i32),
            pltpu.VMEM((CHUNK,), i32),
            pltpu.VMEM((CHUNK,), i32),
            pltpu.VMEM((CHUNK,), i32),
            pltpu.VMEM((CHUNK,), i32),
            pltpu.VMEM((CHUNK,), i32),
            pltpu.VMEM((CHUNK, EMB), f32),
            pltpu.VMEM((CHUNK, EMB), f32),
            pltpu.VMEM((CHUNK, EMB), f32),
            pltpu.VMEM_SHARED((NOUT, EMB), f32),
            pltpu.SemaphoreType.DMA,
            pltpu.SemaphoreType.DMA,
            pltpu.SemaphoreType.DMA,
            pltpu.SemaphoreType.DMA,
            pltpu.SemaphoreType.DMA,
            pltpu.SemaphoreType.DMA,
            pltpu.SemaphoreType.DMA,
            pltpu.SemaphoreType.DMA,
            pltpu.SemaphoreType.DMA,
        ],
        compiler_params=pltpu.CompilerParams(needs_layout_passes=False),
    )
    return kfn(big, idcat, bids, tids, uids, pka, pkb, pkc, pkd)


def _pack(cols, rows, vals):
    i32 = jnp.int32
    nch = PE // CHUNK
    zi = jnp.zeros((PE,), i32)
    c = lax.dynamic_update_slice(zi, cols.astype(i32), (0,)).reshape(nch, CHUNK)
    r = lax.dynamic_update_slice(zi, rows.astype(i32), (0,)).reshape(nch, CHUNK)
    v = lax.dynamic_update_slice(
        zi, lax.bitcast_convert_type(vals, i32), (0,)).reshape(nch, CHUNK)
    return jnp.stack([c, r, v], axis=1).reshape(-1)


def kernel(user_table, item_table, all_top_ids, all_bottom_ids, all_users_ids,
           uj_rows, uj_cols, uj_vals, ij_rows, ij_cols, ij_vals):
    i32 = jnp.int32
    big = jnp.concatenate([item_table, user_table], axis=0)
    tids = all_top_ids.astype(i32)
    bids = all_bottom_ids.astype(i32)
    uids = all_users_ids.astype(i32) + IV

    idcat = jnp.zeros((NCAT,), i32)
    idcat = lax.dynamic_update_slice(idcat, tids, (0,))
    idcat = lax.dynamic_update_slice(idcat, bids, (SEG,))
    idcat = lax.dynamic_update_slice(idcat, uids, (2 * SEG,))

    pka = _pack(uj_cols, uj_rows, uj_vals)
    pkb = _pack(uj_rows, uj_cols, uj_vals)
    pkc = _pack(ij_cols, ij_rows, ij_vals)
    pkd = _pack(ij_rows, ij_cols, ij_vals)

    embcat, u_out, t_out, b_out = _run(big, idcat, bids, tids, uids,
                                       pka, pkb, pkc, pkd)
    top_embs = embcat[0:NT]
    pos_bottoms_embs = embcat[SEG:SEG + NB]
    all_users_embs = embcat[2 * SEG:2 * SEG + NU]
    return (u_out[:NU], t_out[:NT], b_out[:NB],
            top_embs, pos_bottoms_embs, all_users_embs)


# speedup vs baseline: 5.8202x; 1.0100x over previous
import jax
import jax.numpy as jnp
from jax import lax
from jax.experimental import pallas as pl
from jax.experimental.pallas import tpu as pltpu
from jax.experimental.pallas import tpu_sc as plsc

NU = 10000
NT = 10000
NB = 10000
IV = 20000
EMB = 128
NNZ = 320000

NTILES = 16
CHUNK = 96
CPT = 210
PE = CPT * CHUNK * NTILES
PKW = 3 * CHUNK

SEG = 10240
ECPW = 11
NCAT = 32 * ECPW * CHUNK
NOUT = 10240
ROWS_PER_TILE = NOUT // NTILES
FL = 80


def _body(big_hbm, idcat_hbm, bids_hbm, tids_hbm, uids_hbm,
          pka_hbm, pkb_hbm, pkc_hbm, pkd_hbm,
          embcat_hbm, u_hbm, t_hbm, b_hbm,
          ids_v, idx0, idx1, idx2, cidx0, cidx1, cidx2,
          ridx0, ridx1, ridx2, rows0, rows1, rows2, acc,
          gsem0, gsem1, gsem2, ssem0, ssem1, ssem2, isem0, isem1, isem2):
    cid = lax.axis_index("c")
    sid = lax.axis_index("s")
    wid = sid * 2 + cid
    idx_b = (idx0, idx1, idx2)
    cidx_b = (cidx0, cidx1, cidx2)
    ridx_b = (ridx0, ridx1, ridx2)
    rows_b = (rows0, rows1, rows2)
    gsem_b = (gsem0, gsem1, gsem2)
    ssem_b = (ssem0, ssem1, ssem2)
    isem_b = (isem0, isem1, isem2)

    def emb_off(j):
        return (wid * ECPW + j) * CHUNK

    def emb_idx(j, s):
        pltpu.async_copy(idcat_hbm.at[pl.ds(emb_off(j), CHUNK)],
                         cidx_b[s], ssem_b[s])

    def emb_gather(j, s):
        pltpu.make_async_copy(idcat_hbm.at[pl.ds(0, CHUNK)], cidx_b[s],
                              ssem_b[s]).wait()
        pltpu.async_copy(big_hbm.at[cidx_b[s]], rows_b[s], gsem_b[s])

    emb_idx(0, 0)
    emb_idx(1, 1)
    emb_gather(0, 0)
    for j in range(ECPW):
        s = j % 3
        s1 = (j + 1) % 3
        s2 = (j + 2) % 3
        if j + 2 < ECPW:
            emb_idx(j + 2, s2)
        if j + 1 < ECPW:
            if j >= 2:
                pltpu.make_async_copy(rows_b[s1],
                                      embcat_hbm.at[pl.ds(0, CHUNK)],
                                      isem_b[s1]).wait()
            emb_gather(j + 1, s1)
        pltpu.make_async_copy(big_hbm.at[cidx_b[s]], rows_b[s],
                              gsem_b[s]).wait()
        pltpu.async_copy(rows_b[s], embcat_hbm.at[pl.ds(emb_off(j), CHUNK)],
                         isem_b[s])
    for s in range(3):
        if s <= (ECPW - 1) % 3 or ECPW >= 3:
            pltpu.make_async_copy(rows_b[s], embcat_hbm.at[pl.ds(0, CHUNK)],
                                  isem_b[s]).wait()

    base = sid * ROWS_PER_TILE

    def zero_acc():
        def zb(i, carry):
            for g in range(8):
                rows0[i, pl.ds(g * 16, 16)] = jnp.zeros((16,), jnp.float32)
            return carry
        lax.fori_loop(0, CHUNK, zb, 0)
        for j in range(8):
            pltpu.async_copy(rows0.at[pl.ds(0, FL)],
                             acc.at[pl.ds(base + j * FL, FL)], gsem0)
        for j in range(8):
            pltpu.make_async_copy(rows0.at[pl.ds(0, FL)],
                                  acc.at[pl.ds(base, FL)], gsem0).wait()

    def spmm(pk_hbm):
        tile_chunk0 = sid * CPT

        def prep_idx(c, s):
            off = (tile_chunk0 + c) * PKW
            pltpu.async_copy(pk_hbm.at[pl.ds(off, PKW)], idx_b[s], isem_b[s])

        def compose_and_gather(c, s):
            @pl.when(c >= 3)
            def _():
                pltpu.make_async_copy(rows_b[s], acc.at[ridx_b[s]],
                                      ssem_b[s]).wait()
            pltpu.make_async_copy(pk_hbm.at[pl.ds(0, PKW)], idx_b[s],
                                  isem_b[s]).wait()
            for j in range(CHUNK // 16):
                sl = pl.ds(j * 16, 16)
                c16 = idx_b[s][sl]
                cidx_b[s][sl] = plsc.load_gather(ids_v, [c16])
                ridx_b[s][sl] = idx_b[s][pl.ds(CHUNK + j * 16, 16)]
            pltpu.async_copy(big_hbm.at[cidx_b[s]], rows_b[s], gsem_b[s])

        prep_idx(0, 0)
        prep_idx(1, 1)
        compose_and_gather(0, 0)

        def outer(i, carry):
            for k in range(3):
                c = 3 * i + k
                s2 = (k + 2) % 3
                s1 = (k + 1) % 3

                @pl.when(c + 2 < CPT)
                def _():
                    prep_idx(c + 2, s2)

                @pl.when(c + 1 < CPT)
                def _():
                    compose_and_gather(c + 1, s1)

                pltpu.make_async_copy(big_hbm.at[cidx_b[k]],
                                      rows_b[k], gsem_b[k]).wait()

                @plsc.parallel_loop(0, CHUNK, unroll=4)
                def _(e):
                    vb = plsc.bitcast(
                        plsc.load_gather(
                            idx_b[k], [jnp.full((16,), 2 * CHUNK + e,
                                                jnp.int32)]),
                        jnp.float32)
                    for g in range(8):
                        sl = pl.ds(g * 16, 16)
                        rows_b[k][e, sl] = rows_b[k][e, sl] * vb

                pltpu.async_copy(rows_b[k], acc.at[ridx_b[k]], ssem_b[k],
                                 add=True)
            return carry
        lax.fori_loop(0, CPT // 3, outer, 0)
        for s in range(3):
            pltpu.make_async_copy(rows_b[s], acc.at[ridx_b[s]],
                                  ssem_b[s]).wait()

    def flush(out_hbm):
        def frd(j, s):
            pltpu.async_copy(acc.at[pl.ds(base + j * FL, FL)],
                             rows_b[s].at[pl.ds(0, FL)], gsem_b[s])

        frd(0, 0)
        for j in range(8):
            s = j % 2
            s1 = (j + 1) % 2
            if j + 1 < 8:
                if j >= 1:
                    pltpu.make_async_copy(rows_b[s1].at[pl.ds(0, FL)],
                                          out_hbm.at[pl.ds(base, FL)],
                                          ssem_b[s1]).wait()
                frd(j + 1, s1)
            pltpu.make_async_copy(acc.at[pl.ds(base, FL)],
                                  rows_b[s].at[pl.ds(0, FL)],
                                  gsem_b[s]).wait()
            pltpu.async_copy(rows_b[s].at[pl.ds(0, FL)],
                             out_hbm.at[pl.ds(base + j * FL, FL)], ssem_b[s])
        for s in range(2):
            pltpu.make_async_copy(rows_b[s].at[pl.ds(0, FL)],
                                  out_hbm.at[pl.ds(base, FL)],
                                  ssem_b[s]).wait()

    zero_acc()
    plsc.subcore_barrier()

    @pl.when(cid == 0)
    def _():
        pltpu.sync_copy(bids_hbm, ids_v)
        spmm(pka_hbm)

    @pl.when(cid == 1)
    def _():
        pltpu.sync_copy(uids_hbm, ids_v)
        spmm(pkb_hbm)

    plsc.subcore_barrier()

    @pl.when(cid == 0)
    def _():
        flush(u_hbm)

    plsc.subcore_barrier()

    @pl.when(cid == 0)
    def _():
        zero_acc()

    plsc.subcore_barrier()

    @pl.when(cid == 0)
    def _():
        spmm(pkc_hbm)

    @pl.when(cid == 1)
    def _():
        pltpu.sync_copy(tids_hbm, ids_v)
        spmm(pkd_hbm)

    plsc.subcore_barrier()

    @pl.when(cid == 0)
    def _():
        flush(t_hbm)

    @pl.when(cid == 1)
    def _():
        flush(b_hbm)


@jax.jit
def _run(big, idcat, bids, tids, uids, pka, pkb, pkc, pkd):
    f32 = jnp.float32
    i32 = jnp.int32
    mesh = plsc.VectorSubcoreMesh(core_axis_name="c", subcore_axis_name="s")
    kfn = pl.kernel(
        _body,
        out_type=[
            jax.ShapeDtypeStruct((NCAT, EMB), f32),
            jax.ShapeDtypeStruct((NOUT, EMB), f32),
            jax.ShapeDtypeStruct((NOUT, EMB), f32),
            jax.ShapeDtypeStruct((NOUT, EMB), f32),
        ],
        mesh=mesh,
        scratch_types=[
            pltpu.VMEM((NU,), i32),
            pltpu.VMEM((PKW,), i32),
            pltpu.VMEM((PKW,), i32),
            pltpu.VMEM((PKW,), i32),
            pltpu.VMEM((CHUNK,), i32),
            pltpu.VMEM((CHUNK,), i32),
            pltpu.VMEM((CHUNK,), i32),
            pltpu.VMEM((CHUNK,), i32),
            pltpu.VMEM((CHUNK,), i32),
            pltpu.VMEM((CHUNK,), i32),
            pltpu.VMEM((CHUNK, EMB), f32),
            pltpu.VMEM((CHUNK, EMB), f32),
            pltpu.VMEM((CHUNK, EMB), f32),
            pltpu.VMEM_SHARED((NOUT, EMB), f32),
            pltpu.SemaphoreType.DMA,
            pltpu.SemaphoreType.DMA,
            pltpu.SemaphoreType.DMA,
            pltpu.SemaphoreType.DMA,
            pltpu.SemaphoreType.DMA,
            pltpu.SemaphoreType.DMA,
            pltpu.SemaphoreType.DMA,
            pltpu.SemaphoreType.DMA,
            pltpu.SemaphoreType.DMA,
        ],
        compiler_params=pltpu.CompilerParams(needs_layout_passes=False),
    )
    return kfn(big, idcat, bids, tids, uids, pka, pkb, pkc, pkd)


def _pack(cols, rows, vals):
    i32 = jnp.int32
    nch = PE // CHUNK
    zi = jnp.zeros((PE,), i32)
    c = lax.dynamic_update_slice(zi, cols.astype(i32), (0,)).reshape(nch, CHUNK)
    r = lax.dynamic_update_slice(zi, rows.astype(i32), (0,)).reshape(nch, CHUNK)
    v = lax.dynamic_update_slice(
        zi, lax.bitcast_convert_type(vals, i32), (0,)).reshape(nch, CHUNK)
    return jnp.stack([c, r, v], axis=1).reshape(-1)


def kernel(user_table, item_table, all_top_ids, all_bottom_ids, all_users_ids,
           uj_rows, uj_cols, uj_vals, ij_rows, ij_cols, ij_vals):
    i32 = jnp.int32
    big = jnp.concatenate([item_table, user_table], axis=0)
    tids = all_top_ids.astype(i32)
    bids = all_bottom_ids.astype(i32)
    uids = all_users_ids.astype(i32) + IV

    idcat = jnp.zeros((NCAT,), i32)
    idcat = lax.dynamic_update_slice(idcat, tids, (0,))
    idcat = lax.dynamic_update_slice(idcat, bids, (SEG,))
    idcat = lax.dynamic_update_slice(idcat, uids, (2 * SEG,))

    pka = _pack(uj_cols, uj_rows, uj_vals)
    pkb = _pack(uj_rows, uj_cols, uj_vals)
    pkc = _pack(ij_cols, ij_rows, ij_vals)
    pkd = _pack(ij_rows, ij_cols, ij_vals)

    embcat, u_out, t_out, b_out = _run(big, idcat, bids, tids, uids,
                                       pka, pkb, pkc, pkd)
    top_embs = embcat[0:NT]
    pos_bottoms_embs = embcat[SEG:SEG + NB]
    all_users_embs = embcat[2 * SEG:2 * SEG + NU]
    return (u_out[:NU], t_out[:NT], b_out[:NB],
            top_embs, pos_bottoms_embs, all_users_embs)
